# Initial kernel scaffold; baseline (speedup 1.0000x reference)
import jax
import jax.numpy as jnp
from jax.experimental import pallas as pl


def kernel(x, E, W1, b1, W2, b2):
    raise NotImplementedError("write your pallas kernel here")


# speedup vs baseline: 2.5647x; 2.5647x over previous
import functools

import jax
import jax.numpy as jnp
from jax import lax
from jax.experimental import pallas as pl
from jax.experimental.pallas import tpu as pltpu
from jax.experimental.pallas import tpu_sc as plsc

VOCAB = 1000000
EMB = 64
HID = 128
BATCH = 16384
HIST = 50

NC = 2
NS = 16
NW = NC * NS
ROWS_PER_W = BATCH // NW
CB = 2
GROWS = CB * HIST
IDX_STRIDE = 128
CHUNKS = ROWS_PER_W // CB
LANES = 16

_mesh = plsc.VectorSubcoreMesh(core_axis_name="c", subcore_axis_name="s")


@functools.partial(
    pl.kernel,
    out_type=jax.ShapeDtypeStruct((BATCH, EMB), jnp.float32),
    mesh=_mesh,
    scratch_types=[
        pltpu.VMEM((CHUNKS * IDX_STRIDE,), jnp.int32),
        pltpu.VMEM((2, GROWS, EMB), jnp.float32),
        pltpu.VMEM((ROWS_PER_W, EMB), jnp.float32),
        pltpu.SemaphoreType.DMA,
        pltpu.SemaphoreType.DMA,
    ],
    compiler_params=pltpu.CompilerParams(use_tc_tiling_on_sc=False),
)
def _pool_sums(xpad_hbm, e_hbm, out_hbm, xv, gbuf, outv, sem0, sem1):
    wid = lax.axis_index("s") * NC + lax.axis_index("c")
    base = wid * CHUNKS * IDX_STRIDE
    pltpu.sync_copy(xpad_hbm.at[pl.ds(base, CHUNKS * IDX_STRIDE)], xv)

    sems = (sem0, sem1)

    def gather(j, b):
        idx = xv.at[pl.ds(j * IDX_STRIDE, GROWS)]
        return pltpu.make_async_copy(e_hbm.at[idx], gbuf.at[b], sems[b])

    gather(0, 0).start()
    gather(1, 1).start()

    def process(j, b):
        gather(j, b).wait()
        buf = gbuf.at[b]
        for r in range(CB):
            def rbody(i, accs):
                return tuple(
                    accs[k] + buf[r * HIST + i, pl.ds(LANES * k, LANES)]
                    for k in range(EMB // LANES)
                )
            accs = lax.fori_loop(
                0, HIST, rbody,
                tuple(jnp.zeros((LANES,), jnp.float32)
                      for _ in range(EMB // LANES)))
            for k in range(EMB // LANES):
                outv[j * CB + r, pl.ds(LANES * k, LANES)] = accs[k]
        @pl.when(j + 2 < CHUNKS)
        def _():
            gather(j + 2, b).start()

    def body(i, _):
        j = i * 2
        process(j, 0)
        process(j + 1, 1)
        return 0

    lax.fori_loop(0, CHUNKS // 2, body, 0)
    pltpu.sync_copy(outv, out_hbm.at[pl.ds(wid * ROWS_PER_W, ROWS_PER_W)])


def _mlp_body(sums_ref, x_ref, w1_ref, b1_ref, w2_ref, b2_ref, out_ref):
    xb = x_ref[...]
    cnt = jnp.sum((xb > 0).astype(jnp.float32), axis=1, keepdims=True)
    pooled = sums_ref[...] / (cnt + 1e-9)
    h = jnp.maximum(
        jnp.dot(pooled, w1_ref[...], preferred_element_type=jnp.float32)
        + b1_ref[...], 0.0)
    out = (jnp.dot(h, w2_ref[...], preferred_element_type=jnp.float32)
           + b2_ref[...])
    norm = jnp.sqrt(jnp.sum(out * out, axis=1, keepdims=True))
    out_ref[...] = out / jnp.maximum(norm, 1e-12)


_BM = 2048


def _mlp(sums, x, w1, b1, w2, b2):
    return pl.pallas_call(
        _mlp_body,
        grid=(BATCH // _BM,),
        in_specs=[
            pl.BlockSpec((_BM, EMB), lambda i: (i, 0)),
            pl.BlockSpec((_BM, HIST), lambda i: (i, 0)),
            pl.BlockSpec((EMB, HID), lambda i: (0, 0)),
            pl.BlockSpec((1, HID), lambda i: (0, 0)),
            pl.BlockSpec((HID, HID), lambda i: (0, 0)),
            pl.BlockSpec((1, HID), lambda i: (0, 0)),
        ],
        out_specs=pl.BlockSpec((_BM, HID), lambda i: (i, 0)),
        out_shape=jax.ShapeDtypeStruct((BATCH, HID), jnp.float32),
    )(sums, x, w1, b1, w2, b2)


def kernel(x, E, W1, b1, W2, b2):
    xp = jnp.pad(x.reshape(BATCH // CB, CB * HIST).astype(jnp.int32),
                 ((0, 0), (0, IDX_STRIDE - CB * HIST))).reshape(-1)
    sums = _pool_sums(xp, E)
    return _mlp(sums, x.astype(jnp.int32), W1, b1.reshape(1, HID),
                W2, b2.reshape(1, HID))


# speedup vs baseline: 2.7992x; 1.0914x over previous
import functools

import jax
import jax.numpy as jnp
from jax import lax
from jax.experimental import pallas as pl
from jax.experimental.pallas import tpu as pltpu
from jax.experimental.pallas import tpu_sc as plsc

VOCAB = 1000000
EMB = 64
HID = 128
BATCH = 16384
HIST = 50

NC = 2
NS = 16
NW = NC * NS
ROWS_PER_W = BATCH // NW
CB = 2
GROWS = CB * HIST
IDX_STRIDE = 128
CHUNKS = ROWS_PER_W // CB
LANES = 16

_mesh = plsc.VectorSubcoreMesh(core_axis_name="c", subcore_axis_name="s")


@functools.partial(
    pl.kernel,
    out_type=jax.ShapeDtypeStruct((BATCH, EMB), jnp.float32),
    mesh=_mesh,
    scratch_types=[
        pltpu.VMEM((CHUNKS * IDX_STRIDE,), jnp.int32),
        pltpu.VMEM((4, GROWS, EMB), jnp.float32),
        pltpu.VMEM((ROWS_PER_W, EMB), jnp.float32),
        pltpu.SemaphoreType.DMA,
        pltpu.SemaphoreType.DMA,
        pltpu.SemaphoreType.DMA,
        pltpu.SemaphoreType.DMA,
    ],
    compiler_params=pltpu.CompilerParams(use_tc_tiling_on_sc=False),
)
def _pool_sums(xpad_hbm, e_hbm, out_hbm, xv, gbuf, outv,
               sem0, sem1, sem2, sem3):
    wid = lax.axis_index("s") * NC + lax.axis_index("c")
    base = wid * CHUNKS * IDX_STRIDE
    pltpu.sync_copy(xpad_hbm.at[pl.ds(base, CHUNKS * IDX_STRIDE)], xv)

    sems = (sem0, sem1, sem2, sem3)
    NB = 4

    def gather(j, b):
        idx = xv.at[pl.ds(j * IDX_STRIDE, GROWS)]
        return pltpu.make_async_copy(e_hbm.at[idx], gbuf.at[b], sems[b])

    for b in range(NB):
        gather(b, b).start()

    def process(j, b):
        gather(j, b).wait()
        buf = gbuf.at[b]
        for r in range(CB):
            def rbody(i, accs):
                return tuple(
                    accs[k] + buf[r * HIST + i, pl.ds(LANES * k, LANES)]
                    for k in range(EMB // LANES)
                )
            accs = lax.fori_loop(
                0, HIST, rbody,
                tuple(jnp.zeros((LANES,), jnp.float32)
                      for _ in range(EMB // LANES)))
            for k in range(EMB // LANES):
                outv[j * CB + r, pl.ds(LANES * k, LANES)] = accs[k]
        @pl.when(j + NB < CHUNKS)
        def _():
            gather(j + NB, b).start()

    def body(i, _):
        j = i * NB
        for b in range(NB):
            process(j + b, b)
        return 0

    lax.fori_loop(0, CHUNKS // NB, body, 0)
    pltpu.sync_copy(outv, out_hbm.at[pl.ds(wid * ROWS_PER_W, ROWS_PER_W)])


def _mlp_body(sums_ref, x_ref, w1_ref, b1_ref, w2_ref, b2_ref, out_ref):
    xb = x_ref[...]
    cnt = jnp.sum((xb > 0).astype(jnp.float32), axis=1, keepdims=True)
    pooled = sums_ref[...] / (cnt + 1e-9)
    h = jnp.maximum(
        jnp.dot(pooled, w1_ref[...], preferred_element_type=jnp.float32)
        + b1_ref[...], 0.0)
    out = (jnp.dot(h, w2_ref[...], preferred_element_type=jnp.float32)
           + b2_ref[...])
    norm = jnp.sqrt(jnp.sum(out * out, axis=1, keepdims=True))
    out_ref[...] = out / jnp.maximum(norm, 1e-12)


_BM = 2048


def _mlp(sums, x, w1, b1, w2, b2):
    return pl.pallas_call(
        _mlp_body,
        grid=(BATCH // _BM,),
        in_specs=[
            pl.BlockSpec((_BM, EMB), lambda i: (i, 0)),
            pl.BlockSpec((_BM, HIST), lambda i: (i, 0)),
            pl.BlockSpec((EMB, HID), lambda i: (0, 0)),
            pl.BlockSpec((1, HID), lambda i: (0, 0)),
            pl.BlockSpec((HID, HID), lambda i: (0, 0)),
            pl.BlockSpec((1, HID), lambda i: (0, 0)),
        ],
        out_specs=pl.BlockSpec((_BM, HID), lambda i: (i, 0)),
        out_shape=jax.ShapeDtypeStruct((BATCH, HID), jnp.float32),
    )(sums, x, w1, b1, w2, b2)


def kernel(x, E, W1, b1, W2, b2):
    xp = jnp.pad(x.reshape(BATCH // CB, CB * HIST).astype(jnp.int32),
                 ((0, 0), (0, IDX_STRIDE - CB * HIST))).reshape(-1)
    sums = _pool_sums(xp, E)
    return _mlp(sums, x.astype(jnp.int32), W1, b1.reshape(1, HID),
                W2, b2.reshape(1, HID))
